# Initial kernel scaffold; baseline (speedup 1.0000x reference)
import jax
import jax.numpy as jnp
from jax.experimental import pallas as pl


def kernel(x, edge_index, edge_weight, batch, W1, b1, alpha1, gamma1, beta1, W2, b2, alpha2, gamma2, beta2, W3, b3, alpha3, gamma3, beta3, W4, b4, alpha4, gamma4, beta4, W5, b5, alpha5, gamma5, beta5):
    raise NotImplementedError("write your pallas kernel here")


# speedup vs baseline: 2.0512x; 2.0512x over previous
import functools

import jax
import jax.numpy as jnp
from jax import lax
from jax.experimental import pallas as pl
from jax.experimental.pallas import tpu as pltpu
from jax.experimental.pallas import tpu_sc as plsc

N = 10000
E = 160000
G = 64
EPS = 1e-5

EB = 80
EROWS = E // EB
TROWS = EROWS // 16

_mesh = plsc.VectorSubcoreMesh(core_axis_name="c", subcore_axis_name="s")
_sc_params = pltpu.CompilerParams(
    needs_layout_passes=False, use_tc_tiling_on_sc=False)


def _rsqrt(v):
    r = lax.rsqrt(v)
    return r * (1.5 - 0.5 * v * r * r)


@functools.partial(
    pl.kernel,
    mesh=_mesh,
    compiler_params=_sc_params,
    out_type=jax.ShapeDtypeStruct((N, 16), jnp.float32),
    scratch_types=[
        pltpu.VMEM((TROWS, EB), jnp.int32),
        pltpu.VMEM((TROWS, EB), jnp.float32),
        pltpu.VMEM((EB, 16), jnp.float32),
        pltpu.VMEM_SHARED((N, 16), jnp.float32),
    ],
)
def _sc_degree(dst_h, w_h, out_h, dst_v, w_v, rows_v, acc):
    c = lax.axis_index("c")
    s = lax.axis_index("s")

    @pl.when(c == 0)
    def _():
        pltpu.sync_copy(dst_h.at[s], dst_v)
        pltpu.sync_copy(w_h.at[s], w_v)

        def fill1(r, _):
            rows_v[r, :] = jnp.zeros((16,), jnp.float32) + 1.0
            return 0

        lax.fori_loop(0, EB, fill1, 0)

        base = s * 640
        nb = jnp.where(s == 15, 5, 8)

        def initb(b, _):
            pltpu.sync_copy(rows_v, acc.at[pl.ds(base + b * 80, 80)])
            return 0

        lax.fori_loop(0, nb, initb, 0)
        plsc.subcore_barrier()

        def edgeb(j, _):
            def grp(g, _):
                w16 = w_v[j, pl.ds(g * 16, 16)]
                for l in range(16):
                    rows_v[g * 16 + l, :] = (
                        jnp.zeros((16,), jnp.float32) + w16[l])
                return 0

            lax.fori_loop(0, EB // 16, grp, 0)
            pltpu.sync_copy(rows_v, acc.at[dst_v.at[j]], add=True)
            return 0

        lax.fori_loop(0, TROWS, edgeb, 0)
        plsc.subcore_barrier()

        def wbb(b, _):
            start = base + b * 80
            pltpu.sync_copy(acc.at[pl.ds(start, 80)],
                            out_h.at[pl.ds(start, 80)])
            return 0

        lax.fori_loop(0, nb, wbb, 0)


def _make_sc_agg(co2):
    @functools.partial(
        pl.kernel,
        mesh=_mesh,
        compiler_params=_sc_params,
        out_type=jax.ShapeDtypeStruct((co2, N, 64), jnp.float32),
        scratch_types=[
            pltpu.VMEM((TROWS, EB), jnp.int32),
            pltpu.VMEM((TROWS, EB), jnp.int32),
            pltpu.VMEM((TROWS, EB), jnp.float32),
            pltpu.VMEM((EB, 64), jnp.float32),
            pltpu.VMEM_SHARED((N, 64), jnp.float32),
            pltpu.SemaphoreType.DMA,
        ],
    )
    def _sc_agg(xw_h, src_h, dst_h, w_h, out_h,
                src_v, dst_v, w_v, rows_v, acc, sem):
        c = lax.axis_index("c")
        s = lax.axis_index("s")

        pltpu.sync_copy(src_h.at[s], src_v)
        pltpu.sync_copy(dst_h.at[s], dst_v)
        pltpu.sync_copy(w_h.at[s], w_v)
        base = s * 640
        nb = jnp.where(s == 15, 5, 8)

        def chunk_body(cc, k):
            chunk = cc * (co2 // 2) + k
            table = xw_h.at[chunk]

            def initb(b, _):
                start = base + b * 80
                pltpu.sync_copy(table.at[pl.ds(start, 80)], rows_v)
                pltpu.sync_copy(rows_v, acc.at[pl.ds(start, 80)])
                return 0

            lax.fori_loop(0, nb, initb, 0)
            plsc.subcore_barrier()

            def edgeb(j, _):
                pltpu.async_copy(table.at[src_v.at[j]], rows_v, sem).wait()

                def grp(g, _):
                    w16 = w_v[j, pl.ds(g * 16, 16)]
                    for l in range(16):
                        nv = w16[l]
                        r = g * 16 + l
                        for jj in range(4):
                            rows_v[r, pl.ds(jj * 16, 16)] = (
                                rows_v[r, pl.ds(jj * 16, 16)] * nv)
                    return 0

                lax.fori_loop(0, EB // 16, grp, 0)
                pltpu.sync_copy(rows_v, acc.at[dst_v.at[j]], add=True)
                return 0

            lax.fori_loop(0, TROWS, edgeb, 0)
            plsc.subcore_barrier()

            def wbb(b, _):
                start = base + b * 80
                pltpu.sync_copy(acc.at[pl.ds(start, 80)],
                                out_h.at[chunk].at[pl.ds(start, 80)])
                return 0

            lax.fori_loop(0, nb, wbb, 0)
            plsc.subcore_barrier()

        for cc in range(2):
            @pl.when(c == cc)
            def _():
                for k in range(co2 // 2):
                    chunk_body(cc, k)

    return _sc_agg


_sc_agg8 = _make_sc_agg(8)
_sc_agg4 = _make_sc_agg(4)


def _dinv_body(degp_ref, dinv_ref):
    dinv_ref[...] = _rsqrt(degp_ref[...])


def _tc_dinv(degp):
    return pl.pallas_call(
        _dinv_body,
        out_shape=jax.ShapeDtypeStruct((N, 16), jnp.float32),
    )(degp)


def _mm_body(kc, x_ref, w_ref, dinv_ref, o_ref):
    q = pl.program_id(2)
    part = jnp.dot(x_ref[0], w_ref[0, 0], preferred_element_type=jnp.float32)

    @pl.when(q == 0)
    def _():
        o_ref[...] = jnp.zeros_like(o_ref)

    o_ref[...] += part[None]

    @pl.when(q == kc - 1)
    def _():
        o_ref[...] *= dinv_ref[0][None]


def _tc_matmul(h3, W3r, dinv3):
    kc = W3r.shape[0]
    co2 = W3r.shape[1]
    return pl.pallas_call(
        functools.partial(_mm_body, kc),
        grid=(10, co2, kc),
        in_specs=[
            pl.BlockSpec((1, 1000, 64), lambda i, c, q: (q, i, 0)),
            pl.BlockSpec((1, 1, 64, 64), lambda i, c, q: (q, c, 0, 0)),
            pl.BlockSpec((1, 1000, 1), lambda i, c, q: (i, 0, 0)),
        ],
        out_specs=pl.BlockSpec((1, 1000, 64), lambda i, c, q: (c, i, 0)),
        out_shape=jax.ShapeDtypeStruct((co2, N, 64), jnp.float32),
    )(h3, W3r, dinv3)


def _stats_body(agg_ref, b_ref, batch_ref, dinv_ref, sums_ref, sqs_ref, cnt_ref):
    c = pl.program_id(0)
    i = pl.program_id(1)
    z = dinv_ref[0] * agg_ref[0] + b_ref[0]
    bt = batch_ref[0]
    g = lax.broadcasted_iota(jnp.int32, (1, G), 1)
    oh = (bt == g).astype(jnp.float32)
    dn = (((0,), (0,)), ((), ()))
    ssum = lax.dot_general(oh, z, dn, preferred_element_type=jnp.float32,
                  precision=lax.Precision.HIGHEST)
    ssq = lax.dot_general(oh, z * z, dn, preferred_element_type=jnp.float32,
                  precision=lax.Precision.HIGHEST)

    @pl.when(i == 0)
    def _():
        sums_ref[...] = jnp.zeros_like(sums_ref)
        sqs_ref[...] = jnp.zeros_like(sqs_ref)

    sums_ref[...] += ssum[None]
    sqs_ref[...] += ssq[None]

    @pl.when(jnp.logical_and(c == 0, i == 0))
    def _():
        cnt_ref[...] = jnp.zeros_like(cnt_ref)

    @pl.when(c == 0)
    def _():
        cnt_ref[...] += lax.dot_general(
            oh, jnp.ones_like(z), dn, preferred_element_type=jnp.float32,
                  precision=lax.Precision.HIGHEST)


def _tc_stats(agg, b3, batch3, dinv3):
    co2 = agg.shape[0]
    return pl.pallas_call(
        _stats_body,
        grid=(co2, 10),
        in_specs=[
            pl.BlockSpec((1, 1000, 64), lambda c, i: (c, i, 0)),
            pl.BlockSpec((1, 1, 64), lambda c, i: (c, 0, 0)),
            pl.BlockSpec((1, 1000, 1), lambda c, i: (i, 0, 0)),
            pl.BlockSpec((1, 1000, 1), lambda c, i: (i, 0, 0)),
        ],
        out_specs=(
            pl.BlockSpec((1, G, 64), lambda c, i: (c, 0, 0)),
            pl.BlockSpec((1, G, 64), lambda c, i: (c, 0, 0)),
            pl.BlockSpec((G, 64), lambda c, i: (0, 0)),
        ),
        out_shape=(
            jax.ShapeDtypeStruct((co2, G, 64), jnp.float32),
            jax.ShapeDtypeStruct((co2, G, 64), jnp.float32),
            jax.ShapeDtypeStruct((G, 64), jnp.float32),
        ),
    )(agg, b3, batch3, dinv3)


def _norm_body(relu, agg_ref, b_ref, batch_ref, dinv_ref, sums_ref, sqs_ref,
               cnt_ref, alpha_ref, gamma_ref, beta_ref, out_ref):
    z = dinv_ref[0] * agg_ref[0] + b_ref[0]
    n = jnp.maximum(cnt_ref[...], 1.0)
    m = sums_ref[0] / n
    ex2 = sqs_ref[0] / n
    a = alpha_ref[0]
    var = ex2 - (2.0 * a - a * a) * m * m
    inv = _rsqrt(var + EPS)
    bt = batch_ref[0]
    g = lax.broadcasted_iota(jnp.int32, (1, G), 1)
    oh = (bt == g).astype(jnp.float32)
    am_row = jnp.dot(oh, a * m, preferred_element_type=jnp.float32,
                  precision=lax.Precision.HIGHEST)
    inv_row = jnp.dot(oh, inv, preferred_element_type=jnp.float32,
                  precision=lax.Precision.HIGHEST)
    y = gamma_ref[0] * (z - am_row) * inv_row + beta_ref[0]
    if relu:
        y = jnp.maximum(y, 0.0)
    out_ref[...] = y[None]


def _tc_graphnorm(agg, b3, batch3, dinv3, sums, sqs, cnt,
                  alpha3, gamma3, beta3, relu):
    co2 = agg.shape[0]
    return pl.pallas_call(
        functools.partial(_norm_body, relu),
        grid=(10, co2),
        in_specs=[
            pl.BlockSpec((1, 1000, 64), lambda i, c: (c, i, 0)),
            pl.BlockSpec((1, 1, 64), lambda i, c: (c, 0, 0)),
            pl.BlockSpec((1, 1000, 1), lambda i, c: (i, 0, 0)),
            pl.BlockSpec((1, 1000, 1), lambda i, c: (i, 0, 0)),
            pl.BlockSpec((1, G, 64), lambda i, c: (c, 0, 0)),
            pl.BlockSpec((1, G, 64), lambda i, c: (c, 0, 0)),
            pl.BlockSpec((G, 64), lambda i, c: (0, 0)),
            pl.BlockSpec((1, 1, 64), lambda i, c: (c, 0, 0)),
            pl.BlockSpec((1, 1, 64), lambda i, c: (c, 0, 0)),
            pl.BlockSpec((1, 1, 64), lambda i, c: (c, 0, 0)),
        ],
        out_specs=pl.BlockSpec((1, 1000, 64), lambda i, c: (c, i, 0)),
        out_shape=jax.ShapeDtypeStruct((co2, N, 64), jnp.float32),
    )(agg, b3, batch3, dinv3, sums, sqs, cnt, alpha3, gamma3, beta3)


def kernel(x, edge_index, edge_weight, batch,
           W1, b1, alpha1, gamma1, beta1,
           W2, b2, alpha2, gamma2, beta2,
           W3, b3, alpha3, gamma3, beta3,
           W4, b4, alpha4, gamma4, beta4,
           W5, b5, alpha5, gamma5, beta5):
    src = edge_index[0]
    dst = edge_index[1]

    src2 = src.reshape(16, TROWS, EB)
    dst2 = dst.reshape(16, TROWS, EB)
    w2 = edge_weight.reshape(16, TROWS, EB)
    batch3 = batch.reshape(10, 1000, 1)

    degp = _sc_degree(dst2, w2)
    dinv3 = _tc_dinv(degp)[:, :1].reshape(10, 1000, 1)

    h3 = x.reshape(N, 4, 64).transpose(1, 0, 2)
    layers = [
        (W1, b1, alpha1, gamma1, beta1, True),
        (W2, b2, alpha2, gamma2, beta2, True),
        (W3, b3, alpha3, gamma3, beta3, True),
        (W4, b4, alpha4, gamma4, beta4, True),
        (W5, b5, alpha5, gamma5, beta5, False),
    ]
    for (W, b, alpha, gamma, beta, relu) in layers:
        kc = W.shape[0] // 64
        co2 = W.shape[1] // 64
        W3r = W.reshape(kc, 64, co2, 64).transpose(0, 2, 1, 3)
        xw = _tc_matmul(h3, W3r, dinv3)
        agg_fn = _sc_agg8 if co2 == 8 else _sc_agg4
        agg = agg_fn(xw, src2, dst2, w2)
        bc = b.reshape(co2, 1, 64)
        sums, sqs, cnt = _tc_stats(agg, bc, batch3, dinv3)
        h3 = _tc_graphnorm(agg, bc, batch3, dinv3, sums, sqs, cnt,
                           alpha.reshape(co2, 1, 64), gamma.reshape(co2, 1, 64),
                           beta.reshape(co2, 1, 64), relu)
    return h3.transpose(1, 0, 2).reshape(N, 256)


# speedup vs baseline: 3.8968x; 1.8997x over previous
import functools

import jax
import jax.numpy as jnp
from jax import lax
from jax.experimental import pallas as pl
from jax.experimental.pallas import tpu as pltpu
from jax.experimental.pallas import tpu_sc as plsc

N = 10000
E = 160000
G = 64
EPS = 1e-5

EB = 80
EROWS = E // EB
TROWS = EROWS // 16

_mesh = plsc.VectorSubcoreMesh(core_axis_name="c", subcore_axis_name="s")
_sc_params = pltpu.CompilerParams(
    needs_layout_passes=False, use_tc_tiling_on_sc=False)


def _rsqrt(v):
    r = lax.rsqrt(v)
    return r * (1.5 - 0.5 * v * r * r)


@functools.partial(
    pl.kernel,
    mesh=_mesh,
    compiler_params=_sc_params,
    out_type=jax.ShapeDtypeStruct((N, 16), jnp.float32),
    scratch_types=[
        pltpu.VMEM((TROWS, EB), jnp.int32),
        pltpu.VMEM((TROWS, EB), jnp.float32),
        pltpu.VMEM((EB, 16), jnp.float32),
        pltpu.VMEM_SHARED((N, 16), jnp.float32),
    ],
)
def _sc_degree(dst_h, w_h, out_h, dst_v, w_v, rows_v, acc):
    c = lax.axis_index("c")
    s = lax.axis_index("s")

    @pl.when(c == 0)
    def _():
        pltpu.sync_copy(dst_h.at[s], dst_v)
        pltpu.sync_copy(w_h.at[s], w_v)

        def fill1(r, _):
            rows_v[r, :] = jnp.zeros((16,), jnp.float32) + 1.0
            return 0

        lax.fori_loop(0, EB, fill1, 0)

        base = s * 640
        nb = jnp.where(s == 15, 5, 8)

        def initb(b, _):
            pltpu.sync_copy(rows_v, acc.at[pl.ds(base + b * 80, 80)])
            return 0

        lax.fori_loop(0, nb, initb, 0)
        plsc.subcore_barrier()

        def edgeb(j, _):
            def grp(g, _):
                w16 = w_v[j, pl.ds(g * 16, 16)]
                for l in range(16):
                    rows_v[g * 16 + l, :] = (
                        jnp.zeros((16,), jnp.float32) + w16[l])
                return 0

            lax.fori_loop(0, EB // 16, grp, 0)
            pltpu.sync_copy(rows_v, acc.at[dst_v.at[j]], add=True)
            return 0

        lax.fori_loop(0, TROWS, edgeb, 0)
        plsc.subcore_barrier()

        def wbb(b, _):
            start = base + b * 80
            pltpu.sync_copy(acc.at[pl.ds(start, 80)],
                            out_h.at[pl.ds(start, 80)])
            return 0

        lax.fori_loop(0, nb, wbb, 0)


def _make_sc_agg(co2):
    @functools.partial(
        pl.kernel,
        mesh=_mesh,
        compiler_params=_sc_params,
        out_type=jax.ShapeDtypeStruct((co2, N, 64), jnp.float32),
        scratch_types=[
            pltpu.VMEM((TROWS, EB), jnp.int32),
            pltpu.VMEM((TROWS, EB), jnp.int32),
            pltpu.VMEM((TROWS, EB), jnp.float32),
            [pltpu.VMEM((EB, 64), jnp.float32)] * 5,
            [pltpu.VMEM((EB, 64), jnp.float32)] * 5,
            pltpu.VMEM_SHARED((N, 64), jnp.float32),
            [pltpu.SemaphoreType.DMA] * 5,
            [pltpu.SemaphoreType.DMA] * 5,
        ],
    )
    def _sc_agg(xw_h, src_h, dst_h, w_h, out_h,
                src_v, dst_v, w_v, rg, rs, acc, gsem, ssem):
        rows_v = rg[0]
        c = lax.axis_index("c")
        s = lax.axis_index("s")

        pltpu.sync_copy(src_h.at[s], src_v)
        pltpu.sync_copy(dst_h.at[s], dst_v)
        pltpu.sync_copy(w_h.at[s], w_v)
        base = s * 640
        nb = jnp.where(s == 15, 5, 8)

        def chunk_body(cc, k):
            chunk = cc * (co2 // 2) + k
            table = xw_h.at[chunk]

            def initb(b, _):
                start = base + b * 80
                pltpu.sync_copy(table.at[pl.ds(start, 80)], rows_v)
                pltpu.sync_copy(rows_v, acc.at[pl.ds(start, 80)])
                return 0

            lax.fori_loop(0, nb, initb, 0)
            plsc.subcore_barrier()

            for b in range(5):
                pltpu.async_copy(table.at[src_v.at[b]], rg[b], gsem[b])

            def edgeb(t, _):
                for b in range(5):
                    j = t * 5 + b
                    pltpu.make_async_copy(table.at[src_v.at[j]],
                                          rg[b], gsem[b]).wait()

                    @pl.when(t > 0)
                    def _():
                        pltpu.make_async_copy(
                            rs[b], acc.at[dst_v.at[j]], ssem[b]).wait()

                    def grp(g, _):
                        w16 = w_v[j, pl.ds(g * 16, 16)]
                        for l in range(16):
                            nv = w16[l]
                            r = g * 16 + l
                            for jj in range(4):
                                rs[b][r, pl.ds(jj * 16, 16)] = (
                                    rg[b][r, pl.ds(jj * 16, 16)] * nv)
                        return 0

                    lax.fori_loop(0, EB // 16, grp, 0)

                    @pl.when(j + 5 < TROWS)
                    def _():
                        pltpu.async_copy(table.at[src_v.at[j + 5]],
                                         rg[b], gsem[b])

                    pltpu.async_copy(rs[b], acc.at[dst_v.at[j]], ssem[b],
                                     add=True)
                return 0

            lax.fori_loop(0, TROWS // 5, edgeb, 0)
            for b in range(5):
                j_last = TROWS - 5 + b
                pltpu.make_async_copy(rs[b], acc.at[dst_v.at[j_last]],
                                      ssem[b]).wait()
            plsc.subcore_barrier()

            def wbb(b, _):
                start = base + b * 80
                pltpu.sync_copy(acc.at[pl.ds(start, 80)],
                                out_h.at[chunk].at[pl.ds(start, 80)])
                return 0

            lax.fori_loop(0, nb, wbb, 0)
            plsc.subcore_barrier()

        for cc in range(2):
            @pl.when(c == cc)
            def _():
                for k in range(co2 // 2):
                    chunk_body(cc, k)

    return _sc_agg


_sc_agg8 = _make_sc_agg(8)
_sc_agg4 = _make_sc_agg(4)


def _dinv_body(degp_ref, dinv_ref):
    dinv_ref[...] = _rsqrt(degp_ref[...])


def _tc_dinv(degp):
    return pl.pallas_call(
        _dinv_body,
        out_shape=jax.ShapeDtypeStruct((N, 16), jnp.float32),
    )(degp)


def _mm_body(kc, x_ref, w_ref, dinv_ref, o_ref):
    q = pl.program_id(2)
    part = jnp.dot(x_ref[0], w_ref[0, 0], preferred_element_type=jnp.float32)

    @pl.when(q == 0)
    def _():
        o_ref[...] = jnp.zeros_like(o_ref)

    o_ref[...] += part[None]

    @pl.when(q == kc - 1)
    def _():
        o_ref[...] *= dinv_ref[0][None]


def _tc_matmul(h3, W3r, dinv3):
    kc = W3r.shape[0]
    co2 = W3r.shape[1]
    return pl.pallas_call(
        functools.partial(_mm_body, kc),
        grid=(10, co2, kc),
        in_specs=[
            pl.BlockSpec((1, 1000, 64), lambda i, c, q: (q, i, 0)),
            pl.BlockSpec((1, 1, 64, 64), lambda i, c, q: (q, c, 0, 0)),
            pl.BlockSpec((1, 1000, 1), lambda i, c, q: (i, 0, 0)),
        ],
        out_specs=pl.BlockSpec((1, 1000, 64), lambda i, c, q: (c, i, 0)),
        out_shape=jax.ShapeDtypeStruct((co2, N, 64), jnp.float32),
    )(h3, W3r, dinv3)


def _stats_body(agg_ref, b_ref, batch_ref, dinv_ref, sums_ref, sqs_ref, cnt_ref):
    c = pl.program_id(0)
    i = pl.program_id(1)
    z = dinv_ref[0] * agg_ref[0] + b_ref[0]
    bt = batch_ref[0]
    g = lax.broadcasted_iota(jnp.int32, (1, G), 1)
    oh = (bt == g).astype(jnp.float32)
    dn = (((0,), (0,)), ((), ()))
    ssum = lax.dot_general(oh, z, dn, preferred_element_type=jnp.float32,
                  precision=lax.Precision.HIGHEST)
    ssq = lax.dot_general(oh, z * z, dn, preferred_element_type=jnp.float32,
                  precision=lax.Precision.HIGHEST)

    @pl.when(i == 0)
    def _():
        sums_ref[...] = jnp.zeros_like(sums_ref)
        sqs_ref[...] = jnp.zeros_like(sqs_ref)

    sums_ref[...] += ssum[None]
    sqs_ref[...] += ssq[None]

    @pl.when(jnp.logical_and(c == 0, i == 0))
    def _():
        cnt_ref[...] = jnp.zeros_like(cnt_ref)

    @pl.when(c == 0)
    def _():
        cnt_ref[...] += lax.dot_general(
            oh, jnp.ones_like(z), dn, preferred_element_type=jnp.float32,
                  precision=lax.Precision.HIGHEST)


def _tc_stats(agg, b3, batch3, dinv3):
    co2 = agg.shape[0]
    return pl.pallas_call(
        _stats_body,
        grid=(co2, 10),
        in_specs=[
            pl.BlockSpec((1, 1000, 64), lambda c, i: (c, i, 0)),
            pl.BlockSpec((1, 1, 64), lambda c, i: (c, 0, 0)),
            pl.BlockSpec((1, 1000, 1), lambda c, i: (i, 0, 0)),
            pl.BlockSpec((1, 1000, 1), lambda c, i: (i, 0, 0)),
        ],
        out_specs=(
            pl.BlockSpec((1, G, 64), lambda c, i: (c, 0, 0)),
            pl.BlockSpec((1, G, 64), lambda c, i: (c, 0, 0)),
            pl.BlockSpec((G, 64), lambda c, i: (0, 0)),
        ),
        out_shape=(
            jax.ShapeDtypeStruct((co2, G, 64), jnp.float32),
            jax.ShapeDtypeStruct((co2, G, 64), jnp.float32),
            jax.ShapeDtypeStruct((G, 64), jnp.float32),
        ),
    )(agg, b3, batch3, dinv3)


def _norm_body(relu, agg_ref, b_ref, batch_ref, dinv_ref, sums_ref, sqs_ref,
               cnt_ref, alpha_ref, gamma_ref, beta_ref, out_ref):
    z = dinv_ref[0] * agg_ref[0] + b_ref[0]
    n = jnp.maximum(cnt_ref[...], 1.0)
    m = sums_ref[0] / n
    ex2 = sqs_ref[0] / n
    a = alpha_ref[0]
    var = ex2 - (2.0 * a - a * a) * m * m
    inv = _rsqrt(var + EPS)
    bt = batch_ref[0]
    g = lax.broadcasted_iota(jnp.int32, (1, G), 1)
    oh = (bt == g).astype(jnp.float32)
    am_row = jnp.dot(oh, a * m, preferred_element_type=jnp.float32,
                  precision=lax.Precision.HIGHEST)
    inv_row = jnp.dot(oh, inv, preferred_element_type=jnp.float32,
                  precision=lax.Precision.HIGHEST)
    y = gamma_ref[0] * (z - am_row) * inv_row + beta_ref[0]
    if relu:
        y = jnp.maximum(y, 0.0)
    out_ref[...] = y[None]


def _tc_graphnorm(agg, b3, batch3, dinv3, sums, sqs, cnt,
                  alpha3, gamma3, beta3, relu):
    co2 = agg.shape[0]
    return pl.pallas_call(
        functools.partial(_norm_body, relu),
        grid=(10, co2),
        in_specs=[
            pl.BlockSpec((1, 1000, 64), lambda i, c: (c, i, 0)),
            pl.BlockSpec((1, 1, 64), lambda i, c: (c, 0, 0)),
            pl.BlockSpec((1, 1000, 1), lambda i, c: (i, 0, 0)),
            pl.BlockSpec((1, 1000, 1), lambda i, c: (i, 0, 0)),
            pl.BlockSpec((1, G, 64), lambda i, c: (c, 0, 0)),
            pl.BlockSpec((1, G, 64), lambda i, c: (c, 0, 0)),
            pl.BlockSpec((G, 64), lambda i, c: (0, 0)),
            pl.BlockSpec((1, 1, 64), lambda i, c: (c, 0, 0)),
            pl.BlockSpec((1, 1, 64), lambda i, c: (c, 0, 0)),
            pl.BlockSpec((1, 1, 64), lambda i, c: (c, 0, 0)),
        ],
        out_specs=pl.BlockSpec((1, 1000, 64), lambda i, c: (c, i, 0)),
        out_shape=jax.ShapeDtypeStruct((co2, N, 64), jnp.float32),
    )(agg, b3, batch3, dinv3, sums, sqs, cnt, alpha3, gamma3, beta3)


def kernel(x, edge_index, edge_weight, batch,
           W1, b1, alpha1, gamma1, beta1,
           W2, b2, alpha2, gamma2, beta2,
           W3, b3, alpha3, gamma3, beta3,
           W4, b4, alpha4, gamma4, beta4,
           W5, b5, alpha5, gamma5, beta5):
    src = edge_index[0]
    dst = edge_index[1]

    src2 = src.reshape(16, TROWS, EB)
    dst2 = dst.reshape(16, TROWS, EB)
    w2 = edge_weight.reshape(16, TROWS, EB)
    batch3 = batch.reshape(10, 1000, 1)

    degp = _sc_degree(dst2, w2)
    dinv3 = _tc_dinv(degp)[:, :1].reshape(10, 1000, 1)

    h3 = x.reshape(N, 4, 64).transpose(1, 0, 2)
    layers = [
        (W1, b1, alpha1, gamma1, beta1, True),
        (W2, b2, alpha2, gamma2, beta2, True),
        (W3, b3, alpha3, gamma3, beta3, True),
        (W4, b4, alpha4, gamma4, beta4, True),
        (W5, b5, alpha5, gamma5, beta5, False),
    ]
    for (W, b, alpha, gamma, beta, relu) in layers:
        kc = W.shape[0] // 64
        co2 = W.shape[1] // 64
        W3r = W.reshape(kc, 64, co2, 64).transpose(0, 2, 1, 3)
        xw = _tc_matmul(h3, W3r, dinv3)
        agg_fn = _sc_agg8 if co2 == 8 else _sc_agg4
        agg = agg_fn(xw, src2, dst2, w2)
        bc = b.reshape(co2, 1, 64)
        sums, sqs, cnt = _tc_stats(agg, bc, batch3, dinv3)
        h3 = _tc_graphnorm(agg, bc, batch3, dinv3, sums, sqs, cnt,
                           alpha.reshape(co2, 1, 64), gamma.reshape(co2, 1, 64),
                           beta.reshape(co2, 1, 64), relu)
    return h3.transpose(1, 0, 2).reshape(N, 256)


# speedup vs baseline: 5.6642x; 1.4536x over previous
import functools

import jax
import jax.numpy as jnp
from jax import lax
from jax.experimental import pallas as pl
from jax.experimental.pallas import tpu as pltpu
from jax.experimental.pallas import tpu_sc as plsc

N = 10000
E = 160000
G = 64
EPS = 1e-5

EB = 80
EROWS = E // EB
TROWS = EROWS // 16

_mesh = plsc.VectorSubcoreMesh(core_axis_name="c", subcore_axis_name="s")
_sc_params = pltpu.CompilerParams(
    needs_layout_passes=False, use_tc_tiling_on_sc=False)


def _rsqrt(v):
    r = lax.rsqrt(v)
    return r * (1.5 - 0.5 * v * r * r)


@functools.partial(
    pl.kernel,
    mesh=_mesh,
    compiler_params=_sc_params,
    out_type=jax.ShapeDtypeStruct((N, 16), jnp.float32),
    scratch_types=[
        pltpu.VMEM((TROWS, EB), jnp.int32),
        pltpu.VMEM((TROWS, EB), jnp.float32),
        pltpu.VMEM((EB, 16), jnp.float32),
        pltpu.VMEM_SHARED((N, 16), jnp.float32),
    ],
)
def _sc_degree(dst_h, w_h, out_h, dst_v, w_v, rows_v, acc):
    c = lax.axis_index("c")
    s = lax.axis_index("s")

    @pl.when(c == 0)
    def _():
        pltpu.sync_copy(dst_h.at[s], dst_v)
        pltpu.sync_copy(w_h.at[s], w_v)

        def fill1(r, _):
            rows_v[r, :] = jnp.zeros((16,), jnp.float32) + 1.0
            return 0

        lax.fori_loop(0, EB, fill1, 0)

        base = s * 640
        nb = jnp.where(s == 15, 5, 8)

        def initb(b, _):
            pltpu.sync_copy(rows_v, acc.at[pl.ds(base + b * 80, 80)])
            return 0

        lax.fori_loop(0, nb, initb, 0)
        plsc.subcore_barrier()

        def edgeb(j, _):
            def grp(g, _):
                w16 = w_v[j, pl.ds(g * 16, 16)]
                for l in range(16):
                    rows_v[g * 16 + l, :] = (
                        jnp.zeros((16,), jnp.float32) + w16[l])
                return 0

            lax.fori_loop(0, EB // 16, grp, 0)
            pltpu.sync_copy(rows_v, acc.at[dst_v.at[j]], add=True)
            return 0

        lax.fori_loop(0, TROWS, edgeb, 0)
        plsc.subcore_barrier()

        def wbb(b, _):
            start = base + b * 80
            pltpu.sync_copy(acc.at[pl.ds(start, 80)],
                            out_h.at[pl.ds(start, 80)])
            return 0

        lax.fori_loop(0, nb, wbb, 0)


def _make_sc_agg(co2):
    @functools.partial(
        pl.kernel,
        mesh=_mesh,
        compiler_params=_sc_params,
        out_type=jax.ShapeDtypeStruct((co2, N, 64), jnp.float32),
        scratch_types=[
            pltpu.VMEM((TROWS, EB), jnp.int32),
            pltpu.VMEM((TROWS, EB), jnp.int32),
            pltpu.VMEM((TROWS, EB), jnp.float32),
            [pltpu.VMEM((EB, 64), jnp.float32)] * 5,
            [pltpu.VMEM((EB, 64), jnp.float32)] * 5,
            pltpu.VMEM_SHARED((N, 64), jnp.float32),
            [pltpu.SemaphoreType.DMA] * 5,
            [pltpu.SemaphoreType.DMA] * 5,
        ],
    )
    def _sc_agg(xw_h, src_h, dst_h, w_h, out_h,
                src_v, dst_v, w_v, rg, rs, acc, gsem, ssem):
        rows_v = rg[0]
        c = lax.axis_index("c")
        s = lax.axis_index("s")

        pltpu.sync_copy(src_h.at[s], src_v)
        pltpu.sync_copy(dst_h.at[s], dst_v)
        pltpu.sync_copy(w_h.at[s], w_v)
        base = s * 640
        nb = jnp.where(s == 15, 5, 8)

        def chunk_body(cc, k):
            chunk = cc * (co2 // 2) + k
            table = xw_h.at[chunk]

            def initb(b, _):
                start = base + b * 80
                pltpu.sync_copy(table.at[pl.ds(start, 80)], rows_v)
                pltpu.sync_copy(rows_v, acc.at[pl.ds(start, 80)])
                return 0

            lax.fori_loop(0, nb, initb, 0)
            plsc.subcore_barrier()

            for b in range(5):
                pltpu.async_copy(table.at[src_v.at[b]], rg[b], gsem[b])

            def edgeb(t, _):
                for b in range(5):
                    j = t * 5 + b
                    pltpu.make_async_copy(table.at[src_v.at[j]],
                                          rg[b], gsem[b]).wait()

                    @pl.when(t > 0)
                    def _():
                        pltpu.make_async_copy(
                            rs[b], acc.at[dst_v.at[j]], ssem[b]).wait()

                    def grp(g, _):
                        w16 = w_v[j, pl.ds(g * 16, 16)]
                        for l in range(16):
                            nv = w16[l]
                            r = g * 16 + l
                            for jj in range(4):
                                rs[b][r, pl.ds(jj * 16, 16)] = (
                                    rg[b][r, pl.ds(jj * 16, 16)] * nv)
                        return 0

                    lax.fori_loop(0, EB // 16, grp, 0)

                    @pl.when(j + 5 < TROWS)
                    def _():
                        pltpu.async_copy(table.at[src_v.at[j + 5]],
                                         rg[b], gsem[b])

                    pltpu.async_copy(rs[b], acc.at[dst_v.at[j]], ssem[b],
                                     add=True)
                return 0

            lax.fori_loop(0, TROWS // 5, edgeb, 0)
            for b in range(5):
                j_last = TROWS - 5 + b
                pltpu.make_async_copy(rs[b], acc.at[dst_v.at[j_last]],
                                      ssem[b]).wait()
            plsc.subcore_barrier()

            def wbb(b, _):
                start = base + b * 80
                pltpu.sync_copy(acc.at[pl.ds(start, 80)],
                                out_h.at[chunk].at[pl.ds(start, 80)])
                return 0

            lax.fori_loop(0, nb, wbb, 0)
            plsc.subcore_barrier()

        for cc in range(2):
            @pl.when(c == cc)
            def _():
                for k in range(co2 // 2):
                    chunk_body(cc, k)

    return _sc_agg


_sc_agg8 = _make_sc_agg(8)
_sc_agg4 = _make_sc_agg(4)


def _dinv_body(degp_ref, dinv_ref):
    dinv_ref[...] = _rsqrt(degp_ref[...])


def _tc_dinv(degp):
    return pl.pallas_call(
        _dinv_body,
        out_shape=jax.ShapeDtypeStruct((N, 16), jnp.float32),
    )(degp)


def _mm_body(kc, x_ref, w_ref, dinv_ref, o_ref):
    acc = jnp.dot(x_ref[0], w_ref[0, 0], preferred_element_type=jnp.float32)
    for q in range(1, kc):
        acc += jnp.dot(x_ref[q], w_ref[q, 0],
                       preferred_element_type=jnp.float32)
    o_ref[...] = (dinv_ref[0] * acc)[None]


def _tc_matmul(h3, W3r, dinv3):
    kc = W3r.shape[0]
    co2 = W3r.shape[1]
    return pl.pallas_call(
        functools.partial(_mm_body, kc),
        grid=(10, co2),
        in_specs=[
            pl.BlockSpec((kc, 1000, 64), lambda i, c: (0, i, 0)),
            pl.BlockSpec((kc, 1, 64, 64), lambda i, c: (0, c, 0, 0)),
            pl.BlockSpec((1, 1000, 1), lambda i, c: (i, 0, 0)),
        ],
        out_specs=pl.BlockSpec((1, 1000, 64), lambda i, c: (c, i, 0)),
        out_shape=jax.ShapeDtypeStruct((co2, N, 64), jnp.float32),
    )(h3, W3r, dinv3)


def _stats_body(agg_ref, b_ref, batch_ref, dinv_ref, sums_ref, sqs_ref, cnt_ref):
    c = pl.program_id(0)
    i = pl.program_id(1)
    z = dinv_ref[0] * agg_ref[0] + b_ref[0]
    bt = batch_ref[0]
    g = lax.broadcasted_iota(jnp.int32, (1, G), 1)
    oh = (bt == g).astype(jnp.float32)
    dn = (((0,), (0,)), ((), ()))
    ssum = lax.dot_general(oh, z, dn, preferred_element_type=jnp.float32,
                  precision=lax.Precision.HIGHEST)
    ssq = lax.dot_general(oh, z * z, dn, preferred_element_type=jnp.float32,
                  precision=lax.Precision.HIGHEST)

    @pl.when(i == 0)
    def _():
        sums_ref[...] = jnp.zeros_like(sums_ref)
        sqs_ref[...] = jnp.zeros_like(sqs_ref)

    sums_ref[...] += ssum[None]
    sqs_ref[...] += ssq[None]

    @pl.when(jnp.logical_and(c == 0, i == 0))
    def _():
        cnt_ref[...] = jnp.zeros_like(cnt_ref)

    @pl.when(c == 0)
    def _():
        cnt_ref[...] += lax.dot_general(
            oh, jnp.ones_like(z), dn, preferred_element_type=jnp.float32,
                  precision=lax.Precision.HIGHEST)


def _tc_stats(agg, b3, batch3, dinv3):
    co2 = agg.shape[0]
    return pl.pallas_call(
        _stats_body,
        grid=(co2, 10),
        in_specs=[
            pl.BlockSpec((1, 1000, 64), lambda c, i: (c, i, 0)),
            pl.BlockSpec((1, 1, 64), lambda c, i: (c, 0, 0)),
            pl.BlockSpec((1, 1000, 1), lambda c, i: (i, 0, 0)),
            pl.BlockSpec((1, 1000, 1), lambda c, i: (i, 0, 0)),
        ],
        out_specs=(
            pl.BlockSpec((1, G, 64), lambda c, i: (c, 0, 0)),
            pl.BlockSpec((1, G, 64), lambda c, i: (c, 0, 0)),
            pl.BlockSpec((G, 64), lambda c, i: (0, 0)),
        ),
        out_shape=(
            jax.ShapeDtypeStruct((co2, G, 64), jnp.float32),
            jax.ShapeDtypeStruct((co2, G, 64), jnp.float32),
            jax.ShapeDtypeStruct((G, 64), jnp.float32),
        ),
    )(agg, b3, batch3, dinv3)


def _norm_body(relu, agg_ref, b_ref, batch_ref, dinv_ref, sums_ref, sqs_ref,
               cnt_ref, alpha_ref, gamma_ref, beta_ref, out_ref):
    z = dinv_ref[0] * agg_ref[0] + b_ref[0]
    n = jnp.maximum(cnt_ref[...], 1.0)
    m = sums_ref[0] / n
    ex2 = sqs_ref[0] / n
    a = alpha_ref[0]
    var = ex2 - (2.0 * a - a * a) * m * m
    inv = _rsqrt(var + EPS)
    bt = batch_ref[0]
    g = lax.broadcasted_iota(jnp.int32, (1, G), 1)
    oh = (bt == g).astype(jnp.float32)
    am_row = jnp.dot(oh, a * m, preferred_element_type=jnp.float32,
                  precision=lax.Precision.HIGHEST)
    inv_row = jnp.dot(oh, inv, preferred_element_type=jnp.float32,
                  precision=lax.Precision.HIGHEST)
    y = gamma_ref[0] * (z - am_row) * inv_row + beta_ref[0]
    if relu:
        y = jnp.maximum(y, 0.0)
    out_ref[...] = y[None]


def _tc_graphnorm(agg, b3, batch3, dinv3, sums, sqs, cnt,
                  alpha3, gamma3, beta3, relu):
    co2 = agg.shape[0]
    return pl.pallas_call(
        functools.partial(_norm_body, relu),
        grid=(10, co2),
        in_specs=[
            pl.BlockSpec((1, 1000, 64), lambda i, c: (c, i, 0)),
            pl.BlockSpec((1, 1, 64), lambda i, c: (c, 0, 0)),
            pl.BlockSpec((1, 1000, 1), lambda i, c: (i, 0, 0)),
            pl.BlockSpec((1, 1000, 1), lambda i, c: (i, 0, 0)),
            pl.BlockSpec((1, G, 64), lambda i, c: (c, 0, 0)),
            pl.BlockSpec((1, G, 64), lambda i, c: (c, 0, 0)),
            pl.BlockSpec((G, 64), lambda i, c: (0, 0)),
            pl.BlockSpec((1, 1, 64), lambda i, c: (c, 0, 0)),
            pl.BlockSpec((1, 1, 64), lambda i, c: (c, 0, 0)),
            pl.BlockSpec((1, 1, 64), lambda i, c: (c, 0, 0)),
        ],
        out_specs=pl.BlockSpec((1, 1000, 64), lambda i, c: (c, i, 0)),
        out_shape=jax.ShapeDtypeStruct((co2, N, 64), jnp.float32),
    )(agg, b3, batch3, dinv3, sums, sqs, cnt, alpha3, gamma3, beta3)


def kernel(x, edge_index, edge_weight, batch,
           W1, b1, alpha1, gamma1, beta1,
           W2, b2, alpha2, gamma2, beta2,
           W3, b3, alpha3, gamma3, beta3,
           W4, b4, alpha4, gamma4, beta4,
           W5, b5, alpha5, gamma5, beta5):
    src = edge_index[0]
    dst = edge_index[1]

    src2 = src.reshape(16, TROWS, EB)
    dst2 = dst.reshape(16, TROWS, EB)
    w2 = edge_weight.reshape(16, TROWS, EB)
    batch3 = batch.reshape(10, 1000, 1)

    degp = _sc_degree(dst2, w2)
    dinv3 = _tc_dinv(degp)[:, :1].reshape(10, 1000, 1)

    h3 = x.reshape(N, 4, 64).transpose(1, 0, 2)
    layers = [
        (W1, b1, alpha1, gamma1, beta1, True),
        (W2, b2, alpha2, gamma2, beta2, True),
        (W3, b3, alpha3, gamma3, beta3, True),
        (W4, b4, alpha4, gamma4, beta4, True),
        (W5, b5, alpha5, gamma5, beta5, False),
    ]
    for (W, b, alpha, gamma, beta, relu) in layers:
        kc = W.shape[0] // 64
        co2 = W.shape[1] // 64
        W3r = W.reshape(kc, 64, co2, 64).transpose(0, 2, 1, 3)
        xw = _tc_matmul(h3, W3r, dinv3)
        agg_fn = _sc_agg8 if co2 == 8 else _sc_agg4
        agg = agg_fn(xw, src2, dst2, w2)
        bc = b.reshape(co2, 1, 64)
        sums, sqs, cnt = _tc_stats(agg, bc, batch3, dinv3)
        h3 = _tc_graphnorm(agg, bc, batch3, dinv3, sums, sqs, cnt,
                           alpha.reshape(co2, 1, 64), gamma.reshape(co2, 1, 64),
                           beta.reshape(co2, 1, 64), relu)
    return h3.transpose(1, 0, 2).reshape(N, 256)


# speedup vs baseline: 6.1963x; 1.0939x over previous
import functools

import jax
import jax.numpy as jnp
from jax import lax
from jax.experimental import pallas as pl
from jax.experimental.pallas import tpu as pltpu
from jax.experimental.pallas import tpu_sc as plsc

N = 10000
E = 160000
G = 64
EPS = 1e-5

EB = 80
EROWS = E // EB
TROWS = EROWS // 16

_mesh = plsc.VectorSubcoreMesh(core_axis_name="c", subcore_axis_name="s")
_sc_params = pltpu.CompilerParams(
    needs_layout_passes=False, use_tc_tiling_on_sc=False)


def _rsqrt(v):
    r = lax.rsqrt(v)
    return r * (1.5 - 0.5 * v * r * r)


@functools.partial(
    pl.kernel,
    mesh=_mesh,
    compiler_params=_sc_params,
    out_type=jax.ShapeDtypeStruct((N, 16), jnp.float32),
    scratch_types=[
        pltpu.VMEM((TROWS, EB), jnp.int32),
        pltpu.VMEM((TROWS, EB), jnp.float32),
        pltpu.VMEM((EB, 16), jnp.float32),
        pltpu.VMEM_SHARED((N, 16), jnp.float32),
    ],
)
def _sc_degree(dst_h, w_h, out_h, dst_v, w_v, rows_v, acc):
    c = lax.axis_index("c")
    s = lax.axis_index("s")

    @pl.when(c == 0)
    def _():
        pltpu.sync_copy(dst_h.at[s], dst_v)
        pltpu.sync_copy(w_h.at[s], w_v)

        def fill1(r, _):
            rows_v[r, :] = jnp.zeros((16,), jnp.float32) + 1.0
            return 0

        lax.fori_loop(0, EB, fill1, 0)

        base = s * 640
        nb = jnp.where(s == 15, 5, 8)

        def initb(b, _):
            pltpu.sync_copy(rows_v, acc.at[pl.ds(base + b * 80, 80)])
            return 0

        lax.fori_loop(0, nb, initb, 0)
        plsc.subcore_barrier()

        def edgeb(j, _):
            def grp(g, _):
                w16 = w_v[j, pl.ds(g * 16, 16)]
                for l in range(16):
                    rows_v[g * 16 + l, :] = (
                        jnp.zeros((16,), jnp.float32) + w16[l])
                return 0

            lax.fori_loop(0, EB // 16, grp, 0)
            pltpu.sync_copy(rows_v, acc.at[dst_v.at[j]], add=True)
            return 0

        lax.fori_loop(0, TROWS, edgeb, 0)
        plsc.subcore_barrier()

        def wbb(b, _):
            start = base + b * 80
            pltpu.sync_copy(acc.at[pl.ds(start, 80)],
                            out_h.at[pl.ds(start, 80)])
            return 0

        lax.fori_loop(0, nb, wbb, 0)


def _make_sc_agg(co2):
    @functools.partial(
        pl.kernel,
        mesh=_mesh,
        compiler_params=_sc_params,
        out_type=jax.ShapeDtypeStruct((co2, N, 64), jnp.float32),
        scratch_types=[
            pltpu.VMEM((TROWS, EB), jnp.int32),
            pltpu.VMEM((TROWS, EB), jnp.int32),
            pltpu.VMEM((TROWS, EB), jnp.float32),
            [pltpu.VMEM((EB, 64), jnp.float32)] * 5,
            [pltpu.VMEM((EB, 64), jnp.float32)] * 5,
            pltpu.VMEM_SHARED((N, 64), jnp.float32),
            [pltpu.SemaphoreType.DMA] * 5,
            [pltpu.SemaphoreType.DMA] * 5,
        ],
    )
    def _sc_agg(xw_h, src_h, dst_h, w_h, out_h,
                src_v, dst_v, w_v, rg, rs, acc, gsem, ssem):
        rows_v = rg[0]
        c = lax.axis_index("c")
        s = lax.axis_index("s")

        pltpu.sync_copy(src_h.at[s], src_v)
        pltpu.sync_copy(dst_h.at[s], dst_v)
        pltpu.sync_copy(w_h.at[s], w_v)
        base = s * 640
        nb = jnp.where(s == 15, 5, 8)

        def chunk_body(cc, k):
            chunk = cc * (co2 // 2) + k
            table = xw_h.at[chunk]

            def initb(b, _):
                start = base + b * 80
                pltpu.sync_copy(table.at[pl.ds(start, 80)], rows_v)
                pltpu.sync_copy(rows_v, acc.at[pl.ds(start, 80)])
                return 0

            lax.fori_loop(0, nb, initb, 0)
            plsc.subcore_barrier()

            for b in range(5):
                pltpu.async_copy(table.at[src_v.at[b]], rg[b], gsem[b])

            def edgeb(t, _):
                for b in range(5):
                    j = t * 5 + b
                    pltpu.make_async_copy(table.at[src_v.at[j]],
                                          rg[b], gsem[b]).wait()

                    @pl.when(t > 0)
                    def _():
                        pltpu.make_async_copy(
                            rs[b], acc.at[dst_v.at[j]], ssem[b]).wait()

                    def grp(g, _):
                        w16 = w_v[j, pl.ds(g * 16, 16)]
                        for l in range(16):
                            nv = w16[l]
                            r = g * 16 + l
                            for jj in range(4):
                                rs[b][r, pl.ds(jj * 16, 16)] = (
                                    rg[b][r, pl.ds(jj * 16, 16)] * nv)
                        return 0

                    lax.fori_loop(0, EB // 16, grp, 0)

                    @pl.when(j + 5 < TROWS)
                    def _():
                        pltpu.async_copy(table.at[src_v.at[j + 5]],
                                         rg[b], gsem[b])

                    pltpu.async_copy(rs[b], acc.at[dst_v.at[j]], ssem[b],
                                     add=True)
                return 0

            lax.fori_loop(0, TROWS // 5, edgeb, 0)
            for b in range(5):
                j_last = TROWS - 5 + b
                pltpu.make_async_copy(rs[b], acc.at[dst_v.at[j_last]],
                                      ssem[b]).wait()
            plsc.subcore_barrier()

            def wbb(b, _):
                start = base + b * 80
                pltpu.sync_copy(acc.at[pl.ds(start, 80)],
                                out_h.at[chunk].at[pl.ds(start, 80)])
                return 0

            lax.fori_loop(0, nb, wbb, 0)
            plsc.subcore_barrier()

        for cc in range(2):
            @pl.when(c == cc)
            def _():
                for k in range(co2 // 2):
                    chunk_body(cc, k)

    return _sc_agg


_sc_agg8 = _make_sc_agg(8)
_sc_agg4 = _make_sc_agg(4)


def _dinv_body(degp_ref, dinv_ref):
    dinv_ref[...] = _rsqrt(degp_ref[...])


def _tc_dinv(degp):
    return pl.pallas_call(
        _dinv_body,
        out_shape=jax.ShapeDtypeStruct((N, 16), jnp.float32),
    )(degp)


def _mm_body(kc, x_ref, w_ref, dinv_ref, o_ref):
    acc = jnp.dot(x_ref[0], w_ref[0, 0], preferred_element_type=jnp.float32)
    for q in range(1, kc):
        acc += jnp.dot(x_ref[q], w_ref[q, 0],
                       preferred_element_type=jnp.float32)
    o_ref[...] = (dinv_ref[0] * acc)[None]


def _tc_matmul(h3, W3r, dinv3):
    kc = W3r.shape[0]
    co2 = W3r.shape[1]
    return pl.pallas_call(
        functools.partial(_mm_body, kc),
        grid=(10, co2),
        in_specs=[
            pl.BlockSpec((kc, 1000, 64), lambda i, c: (0, i, 0)),
            pl.BlockSpec((kc, 1, 64, 64), lambda i, c: (0, c, 0, 0)),
            pl.BlockSpec((1, 1000, 1), lambda i, c: (i, 0, 0)),
        ],
        out_specs=pl.BlockSpec((1, 1000, 64), lambda i, c: (c, i, 0)),
        out_shape=jax.ShapeDtypeStruct((co2, N, 64), jnp.float32),
    )(h3, W3r, dinv3)


def _stats_body(co2, agg_ref, b_ref, batch_ref, dinv_ref,
                sums_ref, sqs_ref, cnt_ref):
    i = pl.program_id(0)
    bt = batch_ref[0]
    g = lax.broadcasted_iota(jnp.int32, (1, G), 1)
    oh = (bt == g).astype(jnp.float32)
    dn = (((0,), (0,)), ((), ()))

    @pl.when(i == 0)
    def _():
        sums_ref[...] = jnp.zeros_like(sums_ref)
        sqs_ref[...] = jnp.zeros_like(sqs_ref)
        cnt_ref[...] = jnp.zeros_like(cnt_ref)

    dv = dinv_ref[0]
    for c in range(co2):
        z = dv * agg_ref[c] + b_ref[c]
        sums_ref[c] += lax.dot_general(
            oh, z, dn, preferred_element_type=jnp.float32,
            precision=lax.Precision.HIGHEST)
        sqs_ref[c] += lax.dot_general(
            oh, z * z, dn, preferred_element_type=jnp.float32,
            precision=lax.Precision.HIGHEST)
    cnt_ref[...] += lax.dot_general(
        oh, jnp.ones((1000, 64), jnp.float32), dn,
        preferred_element_type=jnp.float32, precision=lax.Precision.HIGHEST)


def _tc_stats(agg, b3, batch3, dinv3):
    co2 = agg.shape[0]
    return pl.pallas_call(
        functools.partial(_stats_body, co2),
        grid=(10,),
        in_specs=[
            pl.BlockSpec((co2, 1000, 64), lambda i: (0, i, 0)),
            pl.BlockSpec((co2, 1, 64), lambda i: (0, 0, 0)),
            pl.BlockSpec((1, 1000, 1), lambda i: (i, 0, 0)),
            pl.BlockSpec((1, 1000, 1), lambda i: (i, 0, 0)),
        ],
        out_specs=(
            pl.BlockSpec((co2, G, 64), lambda i: (0, 0, 0)),
            pl.BlockSpec((co2, G, 64), lambda i: (0, 0, 0)),
            pl.BlockSpec((G, 64), lambda i: (0, 0)),
        ),
        out_shape=(
            jax.ShapeDtypeStruct((co2, G, 64), jnp.float32),
            jax.ShapeDtypeStruct((co2, G, 64), jnp.float32),
            jax.ShapeDtypeStruct((G, 64), jnp.float32),
        ),
    )(agg, b3, batch3, dinv3)


def _norm_body(relu, co2, agg_ref, b_ref, batch_ref, dinv_ref, sums_ref,
               sqs_ref, cnt_ref, alpha_ref, gamma_ref, beta_ref, out_ref):
    n = jnp.maximum(cnt_ref[...], 1.0)
    bt = batch_ref[0]
    g = lax.broadcasted_iota(jnp.int32, (1, G), 1)
    oh = (bt == g).astype(jnp.float32)
    dv = dinv_ref[0]
    for c in range(co2):
        z = dv * agg_ref[c] + b_ref[c]
        m = sums_ref[c] / n
        ex2 = sqs_ref[c] / n
        a = alpha_ref[c]
        var = ex2 - (2.0 * a - a * a) * m * m
        inv = _rsqrt(var + EPS)
        am_row = jnp.dot(oh, a * m, preferred_element_type=jnp.float32,
                         precision=lax.Precision.HIGHEST)
        inv_row = jnp.dot(oh, inv, preferred_element_type=jnp.float32,
                          precision=lax.Precision.HIGHEST)
        y = gamma_ref[c] * (z - am_row) * inv_row + beta_ref[c]
        if relu:
            y = jnp.maximum(y, 0.0)
        out_ref[c] = y


def _tc_graphnorm(agg, b3, batch3, dinv3, sums, sqs, cnt,
                  alpha3, gamma3, beta3, relu):
    co2 = agg.shape[0]
    return pl.pallas_call(
        functools.partial(_norm_body, relu, co2),
        grid=(10,),
        in_specs=[
            pl.BlockSpec((co2, 1000, 64), lambda i: (0, i, 0)),
            pl.BlockSpec((co2, 1, 64), lambda i: (0, 0, 0)),
            pl.BlockSpec((1, 1000, 1), lambda i: (i, 0, 0)),
            pl.BlockSpec((1, 1000, 1), lambda i: (i, 0, 0)),
            pl.BlockSpec((co2, G, 64), lambda i: (0, 0, 0)),
            pl.BlockSpec((co2, G, 64), lambda i: (0, 0, 0)),
            pl.BlockSpec((G, 64), lambda i: (0, 0)),
            pl.BlockSpec((co2, 1, 64), lambda i: (0, 0, 0)),
            pl.BlockSpec((co2, 1, 64), lambda i: (0, 0, 0)),
            pl.BlockSpec((co2, 1, 64), lambda i: (0, 0, 0)),
        ],
        out_specs=pl.BlockSpec((co2, 1000, 64), lambda i: (0, i, 0)),
        out_shape=jax.ShapeDtypeStruct((co2, N, 64), jnp.float32),
    )(agg, b3, batch3, dinv3, sums, sqs, cnt, alpha3, gamma3, beta3)


def kernel(x, edge_index, edge_weight, batch,
           W1, b1, alpha1, gamma1, beta1,
           W2, b2, alpha2, gamma2, beta2,
           W3, b3, alpha3, gamma3, beta3,
           W4, b4, alpha4, gamma4, beta4,
           W5, b5, alpha5, gamma5, beta5):
    src = edge_index[0]
    dst = edge_index[1]

    src2 = src.reshape(16, TROWS, EB)
    dst2 = dst.reshape(16, TROWS, EB)
    w2 = edge_weight.reshape(16, TROWS, EB)
    batch3 = batch.reshape(10, 1000, 1)

    degp = _sc_degree(dst2, w2)
    dinv3 = _tc_dinv(degp)[:, :1].reshape(10, 1000, 1)

    h3 = x.reshape(N, 4, 64).transpose(1, 0, 2)
    layers = [
        (W1, b1, alpha1, gamma1, beta1, True),
        (W2, b2, alpha2, gamma2, beta2, True),
        (W3, b3, alpha3, gamma3, beta3, True),
        (W4, b4, alpha4, gamma4, beta4, True),
        (W5, b5, alpha5, gamma5, beta5, False),
    ]
    for (W, b, alpha, gamma, beta, relu) in layers:
        kc = W.shape[0] // 64
        co2 = W.shape[1] // 64
        W3r = W.reshape(kc, 64, co2, 64).transpose(0, 2, 1, 3)
        xw = _tc_matmul(h3, W3r, dinv3)
        agg_fn = _sc_agg8 if co2 == 8 else _sc_agg4
        agg = agg_fn(xw, src2, dst2, w2)
        bc = b.reshape(co2, 1, 64)
        sums, sqs, cnt = _tc_stats(agg, bc, batch3, dinv3)
        h3 = _tc_graphnorm(agg, bc, batch3, dinv3, sums, sqs, cnt,
                           alpha.reshape(co2, 1, 64), gamma.reshape(co2, 1, 64),
                           beta.reshape(co2, 1, 64), relu)
    return h3.transpose(1, 0, 2).reshape(N, 256)


# speedup vs baseline: 6.5297x; 1.0538x over previous
import functools

import jax
import jax.numpy as jnp
from jax import lax
from jax.experimental import pallas as pl
from jax.experimental.pallas import tpu as pltpu
from jax.experimental.pallas import tpu_sc as plsc

N = 10000
E = 160000
G = 64
EPS = 1e-5

EB = 80
EROWS = E // EB
TROWS = EROWS // 16

_mesh = plsc.VectorSubcoreMesh(core_axis_name="c", subcore_axis_name="s")
_sc_params = pltpu.CompilerParams(
    needs_layout_passes=False, use_tc_tiling_on_sc=False)


def _rsqrt(v):
    r = lax.rsqrt(v)
    return r * (1.5 - 0.5 * v * r * r)


@functools.partial(
    pl.kernel,
    mesh=_mesh,
    compiler_params=_sc_params,
    out_type=jax.ShapeDtypeStruct((N, 16), jnp.float32),
    scratch_types=[
        pltpu.VMEM((TROWS, EB), jnp.int32),
        pltpu.VMEM((TROWS, EB), jnp.float32),
        pltpu.VMEM((EB, 16), jnp.float32),
        pltpu.VMEM_SHARED((N, 16), jnp.float32),
    ],
)
def _sc_degree(dst_h, w_h, out_h, dst_v, w_v, rows_v, acc):
    c = lax.axis_index("c")
    s = lax.axis_index("s")

    @pl.when(c == 0)
    def _():
        pltpu.sync_copy(dst_h.at[s], dst_v)
        pltpu.sync_copy(w_h.at[s], w_v)

        def fill1(r, _):
            rows_v[r, :] = jnp.zeros((16,), jnp.float32) + 1.0
            return 0

        lax.fori_loop(0, EB, fill1, 0)

        base = s * 640
        nb = jnp.where(s == 15, 5, 8)

        def initb(b, _):
            pltpu.sync_copy(rows_v, acc.at[pl.ds(base + b * 80, 80)])
            return 0

        lax.fori_loop(0, nb, initb, 0)
        plsc.subcore_barrier()

        def edgeb(j, _):
            def grp(g, _):
                w16 = w_v[j, pl.ds(g * 16, 16)]
                for l in range(16):
                    rows_v[g * 16 + l, :] = (
                        jnp.zeros((16,), jnp.float32) + w16[l])
                return 0

            lax.fori_loop(0, EB // 16, grp, 0)
            pltpu.sync_copy(rows_v, acc.at[dst_v.at[j]], add=True)
            return 0

        lax.fori_loop(0, TROWS, edgeb, 0)
        plsc.subcore_barrier()

        def wbb(b, _):
            start = base + b * 80
            pltpu.sync_copy(acc.at[pl.ds(start, 80)],
                            out_h.at[pl.ds(start, 80)])
            return 0

        lax.fori_loop(0, nb, wbb, 0)


def _make_sc_agg(co2):
    @functools.partial(
        pl.kernel,
        mesh=_mesh,
        compiler_params=_sc_params,
        out_type=jax.ShapeDtypeStruct((co2, N, 64), jnp.float32),
        scratch_types=[
            pltpu.VMEM((TROWS, EB), jnp.int32),
            pltpu.VMEM((TROWS, EB), jnp.int32),
            pltpu.VMEM((TROWS, EB), jnp.float32),
            [pltpu.VMEM((EB, 64), jnp.float32)] * 5,
            [pltpu.VMEM((EB, 64), jnp.float32)] * 5,
            pltpu.VMEM_SHARED((N, 64), jnp.float32),
            [pltpu.SemaphoreType.DMA] * 5,
            [pltpu.SemaphoreType.DMA] * 5,
        ],
    )
    def _sc_agg(xw_h, src_h, dst_h, w_h, out_h,
                src_v, dst_v, w_v, rg, rs, acc, gsem, ssem):
        rows_v = rg[0]
        c = lax.axis_index("c")
        s = lax.axis_index("s")

        pltpu.sync_copy(src_h.at[s], src_v)
        pltpu.sync_copy(dst_h.at[s], dst_v)
        pltpu.sync_copy(w_h.at[s], w_v)
        base = s * 640
        nb = jnp.where(s == 15, 5, 8)

        def chunk_body(cc, k):
            chunk = cc * (co2 // 2) + k
            table = xw_h.at[chunk]

            def initb(b, _):
                start = base + b * 80
                pltpu.sync_copy(table.at[pl.ds(start, 80)], rows_v)
                pltpu.sync_copy(rows_v, acc.at[pl.ds(start, 80)])
                return 0

            lax.fori_loop(0, nb, initb, 0)
            plsc.subcore_barrier()

            for b in range(5):
                pltpu.async_copy(table.at[src_v.at[b]], rg[b], gsem[b])

            def edgeb(t, _):
                for b in range(5):
                    j = t * 5 + b
                    pltpu.make_async_copy(table.at[src_v.at[j]],
                                          rg[b], gsem[b]).wait()

                    @pl.when(t > 0)
                    def _():
                        pltpu.make_async_copy(
                            rs[b], acc.at[dst_v.at[j]], ssem[b]).wait()

                    def grp(g, _):
                        w16 = w_v[j, pl.ds(g * 16, 16)]
                        for l in range(16):
                            nv = w16[l]
                            r = g * 16 + l
                            for jj in range(4):
                                rs[b][r, pl.ds(jj * 16, 16)] = (
                                    rg[b][r, pl.ds(jj * 16, 16)] * nv)
                        return 0

                    lax.fori_loop(0, EB // 16, grp, 0)

                    @pl.when(j + 5 < TROWS)
                    def _():
                        pltpu.async_copy(table.at[src_v.at[j + 5]],
                                         rg[b], gsem[b])

                    pltpu.async_copy(rs[b], acc.at[dst_v.at[j]], ssem[b],
                                     add=True)
                return 0

            lax.fori_loop(0, TROWS // 5, edgeb, 0)
            for b in range(5):
                j_last = TROWS - 5 + b
                pltpu.make_async_copy(rs[b], acc.at[dst_v.at[j_last]],
                                      ssem[b]).wait()
            plsc.subcore_barrier()

            def wbb(b, _):
                start = base + b * 80
                pltpu.sync_copy(acc.at[pl.ds(start, 80)],
                                out_h.at[chunk].at[pl.ds(start, 80)])
                return 0

            lax.fori_loop(0, nb, wbb, 0)
            plsc.subcore_barrier()

        for cc in range(2):
            @pl.when(c == cc)
            def _():
                for k in range(co2 // 2):
                    chunk_body(cc, k)

    return _sc_agg


_sc_agg8 = _make_sc_agg(8)
_sc_agg4 = _make_sc_agg(4)


def _dinv_body(degp_ref, dinv_ref):
    dinv_ref[...] = _rsqrt(degp_ref[...])


def _tc_dinv(degp):
    return pl.pallas_call(
        _dinv_body,
        out_shape=jax.ShapeDtypeStruct((N, 16), jnp.float32),
    )(degp)


def _mm_body(kc, x_ref, w_ref, dinv_ref, o_ref):
    acc = jnp.dot(x_ref[0], w_ref[0, 0], preferred_element_type=jnp.float32)
    for q in range(1, kc):
        acc += jnp.dot(x_ref[q], w_ref[q, 0],
                       preferred_element_type=jnp.float32)
    o_ref[...] = (dinv_ref[0] * acc)[None]


def _tc_matmul(h3, W3r, dinv3):
    kc = W3r.shape[0]
    co2 = W3r.shape[1]
    return pl.pallas_call(
        functools.partial(_mm_body, kc),
        grid=(10, co2),
        in_specs=[
            pl.BlockSpec((kc, 1000, 64), lambda i, c: (0, i, 0)),
            pl.BlockSpec((kc, 1, 64, 64), lambda i, c: (0, c, 0, 0)),
            pl.BlockSpec((1, 1000, 1), lambda i, c: (i, 0, 0)),
        ],
        out_specs=pl.BlockSpec((1, 1000, 64), lambda i, c: (c, i, 0)),
        out_shape=jax.ShapeDtypeStruct((co2, N, 64), jnp.float32),
    )(h3, W3r, dinv3)


def _stats_body(co2, agg_ref, b_ref, batch_ref, dinv_ref,
                sums_ref, sqs_ref, cnt_ref):
    i = pl.program_id(0)
    bt = batch_ref[0]
    g = lax.broadcasted_iota(jnp.int32, (1, G), 1)
    oh = (bt == g).astype(jnp.float32)
    dn = (((0,), (0,)), ((), ()))

    @pl.when(i == 0)
    def _():
        sums_ref[...] = jnp.zeros_like(sums_ref)
        sqs_ref[...] = jnp.zeros_like(sqs_ref)
        cnt_ref[...] = jnp.zeros_like(cnt_ref)

    dv = dinv_ref[0]
    for c in range(co2):
        z = dv * agg_ref[c] + b_ref[c]
        sums_ref[c] += lax.dot_general(
            oh, z, dn, preferred_element_type=jnp.float32,
            precision=lax.Precision.HIGHEST)
        sqs_ref[c] += lax.dot_general(
            oh, z * z, dn, preferred_element_type=jnp.float32,
            precision=lax.Precision.HIGHEST)
    cnt_ref[...] += lax.dot_general(
        oh, jnp.ones((1000, 64), jnp.float32), dn,
        preferred_element_type=jnp.float32, precision=lax.Precision.HIGHEST)


def _tc_stats(agg, b3, batch3, dinv3):
    co2 = agg.shape[0]
    return pl.pallas_call(
        functools.partial(_stats_body, co2),
        grid=(10,),
        in_specs=[
            pl.BlockSpec((co2, 1000, 64), lambda i: (0, i, 0)),
            pl.BlockSpec((co2, 1, 64), lambda i: (0, 0, 0)),
            pl.BlockSpec((1, 1000, 1), lambda i: (i, 0, 0)),
            pl.BlockSpec((1, 1000, 1), lambda i: (i, 0, 0)),
        ],
        out_specs=(
            pl.BlockSpec((co2, G, 64), lambda i: (0, 0, 0)),
            pl.BlockSpec((co2, G, 64), lambda i: (0, 0, 0)),
            pl.BlockSpec((G, 64), lambda i: (0, 0)),
        ),
        out_shape=(
            jax.ShapeDtypeStruct((co2, G, 64), jnp.float32),
            jax.ShapeDtypeStruct((co2, G, 64), jnp.float32),
            jax.ShapeDtypeStruct((G, 64), jnp.float32),
        ),
    )(agg, b3, batch3, dinv3)


def _norm_rows(co2, agg_ref, b_ref, batch_ref, dinv_ref, sums_ref,
               sqs_ref, cnt_ref, alpha_ref, gamma_ref, beta_ref, relu):
    n = jnp.maximum(cnt_ref[...], 1.0)
    bt = batch_ref[0]
    g = lax.broadcasted_iota(jnp.int32, (1, G), 1)
    oh = (bt == g).astype(jnp.float32)
    dv = dinv_ref[0]
    ys = []
    for c in range(co2):
        z = dv * agg_ref[c] + b_ref[c]
        m = sums_ref[c] / n
        ex2 = sqs_ref[c] / n
        a = alpha_ref[c]
        var = ex2 - (2.0 * a - a * a) * m * m
        inv = _rsqrt(var + EPS)
        am_row = jnp.dot(oh, a * m, preferred_element_type=jnp.float32,
                         precision=lax.Precision.HIGHEST)
        inv_row = jnp.dot(oh, inv, preferred_element_type=jnp.float32,
                          precision=lax.Precision.HIGHEST)
        y = gamma_ref[c] * (z - am_row) * inv_row + beta_ref[c]
        if relu:
            y = jnp.maximum(y, 0.0)
        ys.append(y)
    return ys


def _norm_mm_body(co2p, co2n, agg_ref, b_ref, batch_ref, dinv_ref, sums_ref,
                  sqs_ref, cnt_ref, alpha_ref, gamma_ref, beta_ref, w_ref,
                  o_ref):
    ys = _norm_rows(co2p, agg_ref, b_ref, batch_ref, dinv_ref, sums_ref,
                    sqs_ref, cnt_ref, alpha_ref, gamma_ref, beta_ref, True)
    dv = dinv_ref[0]
    for co in range(co2n):
        acc = jnp.dot(ys[0], w_ref[0, co], preferred_element_type=jnp.float32)
        for q in range(1, co2p):
            acc += jnp.dot(ys[q], w_ref[q, co],
                           preferred_element_type=jnp.float32)
        o_ref[co] = dv * acc


def _tc_norm_mm(agg, b3, batch3, dinv3, sums, sqs, cnt,
                alpha3, gamma3, beta3, W3r):
    co2p = agg.shape[0]
    co2n = W3r.shape[1]
    return pl.pallas_call(
        functools.partial(_norm_mm_body, co2p, co2n),
        grid=(10,),
        in_specs=[
            pl.BlockSpec((co2p, 1000, 64), lambda i: (0, i, 0)),
            pl.BlockSpec((co2p, 1, 64), lambda i: (0, 0, 0)),
            pl.BlockSpec((1, 1000, 1), lambda i: (i, 0, 0)),
            pl.BlockSpec((1, 1000, 1), lambda i: (i, 0, 0)),
            pl.BlockSpec((co2p, G, 64), lambda i: (0, 0, 0)),
            pl.BlockSpec((co2p, G, 64), lambda i: (0, 0, 0)),
            pl.BlockSpec((G, 64), lambda i: (0, 0)),
            pl.BlockSpec((co2p, 1, 64), lambda i: (0, 0, 0)),
            pl.BlockSpec((co2p, 1, 64), lambda i: (0, 0, 0)),
            pl.BlockSpec((co2p, 1, 64), lambda i: (0, 0, 0)),
            pl.BlockSpec((co2p, co2n, 64, 64), lambda i: (0, 0, 0, 0)),
        ],
        out_specs=pl.BlockSpec((co2n, 1000, 64), lambda i: (0, i, 0)),
        out_shape=jax.ShapeDtypeStruct((co2n, N, 64), jnp.float32),
    )(agg, b3, batch3, dinv3, sums, sqs, cnt, alpha3, gamma3, beta3, W3r)


def _norm_final_body(co2, agg_ref, b_ref, batch_ref, dinv_ref, sums_ref,
                     sqs_ref, cnt_ref, alpha_ref, gamma_ref, beta_ref,
                     out_ref):
    ys = _norm_rows(co2, agg_ref, b_ref, batch_ref, dinv_ref, sums_ref,
                    sqs_ref, cnt_ref, alpha_ref, gamma_ref, beta_ref, False)
    for c in range(co2):
        out_ref[:, pl.ds(c * 64, 64)] = ys[c]


def _tc_norm_final(agg, b3, batch3, dinv3, sums, sqs, cnt,
                   alpha3, gamma3, beta3):
    co2 = agg.shape[0]
    return pl.pallas_call(
        functools.partial(_norm_final_body, co2),
        grid=(10,),
        in_specs=[
            pl.BlockSpec((co2, 1000, 64), lambda i: (0, i, 0)),
            pl.BlockSpec((co2, 1, 64), lambda i: (0, 0, 0)),
            pl.BlockSpec((1, 1000, 1), lambda i: (i, 0, 0)),
            pl.BlockSpec((1, 1000, 1), lambda i: (i, 0, 0)),
            pl.BlockSpec((co2, G, 64), lambda i: (0, 0, 0)),
            pl.BlockSpec((co2, G, 64), lambda i: (0, 0, 0)),
            pl.BlockSpec((G, 64), lambda i: (0, 0)),
            pl.BlockSpec((co2, 1, 64), lambda i: (0, 0, 0)),
            pl.BlockSpec((co2, 1, 64), lambda i: (0, 0, 0)),
            pl.BlockSpec((co2, 1, 64), lambda i: (0, 0, 0)),
        ],
        out_specs=pl.BlockSpec((1000, 64 * co2), lambda i: (i, 0)),
        out_shape=jax.ShapeDtypeStruct((N, 64 * co2), jnp.float32),
    )(agg, b3, batch3, dinv3, sums, sqs, cnt, alpha3, gamma3, beta3)


def kernel(x, edge_index, edge_weight, batch,
           W1, b1, alpha1, gamma1, beta1,
           W2, b2, alpha2, gamma2, beta2,
           W3, b3, alpha3, gamma3, beta3,
           W4, b4, alpha4, gamma4, beta4,
           W5, b5, alpha5, gamma5, beta5):
    src = edge_index[0]
    dst = edge_index[1]

    src2 = src.reshape(16, TROWS, EB)
    dst2 = dst.reshape(16, TROWS, EB)
    w2 = edge_weight.reshape(16, TROWS, EB)
    batch3 = batch.reshape(10, 1000, 1)

    degp = _sc_degree(dst2, w2)
    dinv3 = _tc_dinv(degp)[:, :1].reshape(10, 1000, 1)

    h3 = x.reshape(N, 4, 64).transpose(1, 0, 2)
    layers = [
        (W1, b1, alpha1, gamma1, beta1),
        (W2, b2, alpha2, gamma2, beta2),
        (W3, b3, alpha3, gamma3, beta3),
        (W4, b4, alpha4, gamma4, beta4),
        (W5, b5, alpha5, gamma5, beta5),
    ]

    def wres(W):
        kc = W.shape[0] // 64
        co2 = W.shape[1] // 64
        return W.reshape(kc, 64, co2, 64).transpose(0, 2, 1, 3)

    xw = _tc_matmul(h3, wres(W1), dinv3)
    for li, (W, b, alpha, gamma, beta) in enumerate(layers):
        co2 = W.shape[1] // 64
        agg_fn = _sc_agg8 if co2 == 8 else _sc_agg4
        agg = agg_fn(xw, src2, dst2, w2)
        bc = b.reshape(co2, 1, 64)
        sums, sqs, cnt = _tc_stats(agg, bc, batch3, dinv3)
        params = (bc, batch3, dinv3, sums, sqs, cnt,
                  alpha.reshape(co2, 1, 64), gamma.reshape(co2, 1, 64),
                  beta.reshape(co2, 1, 64))
        if li < 4:
            xw = _tc_norm_mm(agg, *params, wres(layers[li + 1][0]))
        else:
            out = _tc_norm_final(agg, *params)
    return out


# speedup vs baseline: 6.5642x; 1.0053x over previous
import functools

import jax
import jax.numpy as jnp
from jax import lax
from jax.experimental import pallas as pl
from jax.experimental.pallas import tpu as pltpu
from jax.experimental.pallas import tpu_sc as plsc

N = 10000
E = 160000
G = 64
EPS = 1e-5

EB = 80
EROWS = E // EB
TROWS = EROWS // 16

_mesh = plsc.VectorSubcoreMesh(core_axis_name="c", subcore_axis_name="s")
_sc_params = pltpu.CompilerParams(
    needs_layout_passes=False, use_tc_tiling_on_sc=False)


def _rsqrt(v):
    r = lax.rsqrt(v)
    return r * (1.5 - 0.5 * v * r * r)


@functools.partial(
    pl.kernel,
    mesh=_mesh,
    compiler_params=_sc_params,
    out_type=jax.ShapeDtypeStruct((N, 16), jnp.float32),
    scratch_types=[
        pltpu.VMEM((TROWS, EB), jnp.int32),
        pltpu.VMEM((TROWS, EB), jnp.float32),
        [pltpu.VMEM((EB, 16), jnp.float32)] * 5,
        pltpu.VMEM_SHARED((N, 16), jnp.float32),
        [pltpu.SemaphoreType.DMA] * 5,
    ],
)
def _sc_degree(dst_h, w_h, out_h, dst_v, w_v, rbufs, acc, ssem):
    rows_v = rbufs[0]
    c = lax.axis_index("c")
    s = lax.axis_index("s")

    @pl.when(c == 0)
    def _():
        pltpu.sync_copy(dst_h.at[s], dst_v)
        pltpu.sync_copy(w_h.at[s], w_v)

        def fill1(r, _):
            rows_v[r, :] = jnp.zeros((16,), jnp.float32) + 1.0
            return 0

        lax.fori_loop(0, EB, fill1, 0)

        base = s * 640
        nb = jnp.where(s == 15, 5, 8)

        def initb(b, _):
            pltpu.sync_copy(rows_v, acc.at[pl.ds(base + b * 80, 80)])
            return 0

        lax.fori_loop(0, nb, initb, 0)
        plsc.subcore_barrier()

        def edgeb(t, _):
            for b in range(5):
                j = t * 5 + b

                @pl.when(t > 0)
                def _():
                    pltpu.make_async_copy(
                        rbufs[b], acc.at[dst_v.at[j]], ssem[b]).wait()

                def grp(g, _):
                    w16 = w_v[j, pl.ds(g * 16, 16)]
                    for l in range(16):
                        rbufs[b][g * 16 + l, :] = (
                            jnp.zeros((16,), jnp.float32) + w16[l])
                    return 0

                lax.fori_loop(0, EB // 16, grp, 0)
                pltpu.async_copy(rbufs[b], acc.at[dst_v.at[j]], ssem[b],
                                 add=True)
            return 0

        lax.fori_loop(0, TROWS // 5, edgeb, 0)
        for b in range(5):
            pltpu.make_async_copy(rbufs[b], acc.at[dst_v.at[TROWS - 5 + b]],
                                  ssem[b]).wait()
        plsc.subcore_barrier()

        def wbb(b, _):
            start = base + b * 80
            pltpu.sync_copy(acc.at[pl.ds(start, 80)],
                            out_h.at[pl.ds(start, 80)])
            return 0

        lax.fori_loop(0, nb, wbb, 0)


def _make_sc_agg(co2):
    @functools.partial(
        pl.kernel,
        mesh=_mesh,
        compiler_params=_sc_params,
        out_type=jax.ShapeDtypeStruct((co2, N, 64), jnp.float32),
        scratch_types=[
            pltpu.VMEM((TROWS, EB), jnp.int32),
            pltpu.VMEM((TROWS, EB), jnp.int32),
            pltpu.VMEM((TROWS, EB), jnp.float32),
            [pltpu.VMEM((EB, 64), jnp.float32)] * 5,
            [pltpu.VMEM((EB, 64), jnp.float32)] * 5,
            pltpu.VMEM_SHARED((N, 64), jnp.float32),
            [pltpu.SemaphoreType.DMA] * 5,
            [pltpu.SemaphoreType.DMA] * 5,
        ],
    )
    def _sc_agg(xw_h, src_h, dst_h, w_h, out_h,
                src_v, dst_v, w_v, rg, rs, acc, gsem, ssem):
        rows_v = rg[0]
        c = lax.axis_index("c")
        s = lax.axis_index("s")

        pltpu.sync_copy(src_h.at[s], src_v)
        pltpu.sync_copy(dst_h.at[s], dst_v)
        pltpu.sync_copy(w_h.at[s], w_v)
        base = s * 640
        nb = jnp.where(s == 15, 5, 8)

        def chunk_body(cc, k):
            chunk = cc * (co2 // 2) + k
            table = xw_h.at[chunk]

            def initb(b, _):
                start = base + b * 80
                pltpu.sync_copy(table.at[pl.ds(start, 80)], rows_v)
                pltpu.sync_copy(rows_v, acc.at[pl.ds(start, 80)])
                return 0

            lax.fori_loop(0, nb, initb, 0)
            plsc.subcore_barrier()

            for b in range(5):
                pltpu.async_copy(table.at[src_v.at[b]], rg[b], gsem[b])

            def edgeb(t, _):
                for b in range(5):
                    j = t * 5 + b
                    pltpu.make_async_copy(table.at[src_v.at[j]],
                                          rg[b], gsem[b]).wait()

                    @pl.when(t > 0)
                    def _():
                        pltpu.make_async_copy(
                            rs[b], acc.at[dst_v.at[j]], ssem[b]).wait()

                    def grp(g, _):
                        w16 = w_v[j, pl.ds(g * 16, 16)]
                        for l in range(16):
                            nv = w16[l]
                            r = g * 16 + l
                            for jj in range(4):
                                rs[b][r, pl.ds(jj * 16, 16)] = (
                                    rg[b][r, pl.ds(jj * 16, 16)] * nv)
                        return 0

                    lax.fori_loop(0, EB // 16, grp, 0)

                    @pl.when(j + 5 < TROWS)
                    def _():
                        pltpu.async_copy(table.at[src_v.at[j + 5]],
                                         rg[b], gsem[b])

                    pltpu.async_copy(rs[b], acc.at[dst_v.at[j]], ssem[b],
                                     add=True)
                return 0

            lax.fori_loop(0, TROWS // 5, edgeb, 0)
            for b in range(5):
                j_last = TROWS - 5 + b
                pltpu.make_async_copy(rs[b], acc.at[dst_v.at[j_last]],
                                      ssem[b]).wait()
            plsc.subcore_barrier()

            def wbb(b, _):
                start = base + b * 80
                pltpu.sync_copy(acc.at[pl.ds(start, 80)],
                                out_h.at[chunk].at[pl.ds(start, 80)])
                return 0

            lax.fori_loop(0, nb, wbb, 0)
            plsc.subcore_barrier()

        for cc in range(2):
            @pl.when(c == cc)
            def _():
                for k in range(co2 // 2):
                    chunk_body(cc, k)

    return _sc_agg


_sc_agg8 = _make_sc_agg(8)
_sc_agg4 = _make_sc_agg(4)


def _dinv_body(degp_ref, dinv_ref):
    dinv_ref[...] = _rsqrt(degp_ref[...])


def _tc_dinv(degp):
    return pl.pallas_call(
        _dinv_body,
        out_shape=jax.ShapeDtypeStruct((N, 16), jnp.float32),
    )(degp)


def _mm_body(kc, x_ref, w_ref, dinv_ref, o_ref):
    acc = jnp.dot(x_ref[0], w_ref[0, 0], preferred_element_type=jnp.float32)
    for q in range(1, kc):
        acc += jnp.dot(x_ref[q], w_ref[q, 0],
                       preferred_element_type=jnp.float32)
    o_ref[...] = (dinv_ref[0] * acc)[None]


def _tc_matmul(h3, W3r, dinv3):
    kc = W3r.shape[0]
    co2 = W3r.shape[1]
    return pl.pallas_call(
        functools.partial(_mm_body, kc),
        grid=(10, co2),
        in_specs=[
            pl.BlockSpec((kc, 1000, 64), lambda i, c: (0, i, 0)),
            pl.BlockSpec((kc, 1, 64, 64), lambda i, c: (0, c, 0, 0)),
            pl.BlockSpec((1, 1000, 1), lambda i, c: (i, 0, 0)),
        ],
        out_specs=pl.BlockSpec((1, 1000, 64), lambda i, c: (c, i, 0)),
        out_shape=jax.ShapeDtypeStruct((co2, N, 64), jnp.float32),
    )(h3, W3r, dinv3)


def _stats_body(co2, agg_ref, b_ref, batch_ref, dinv_ref,
                sums_ref, sqs_ref, cnt_ref):
    i = pl.program_id(0)
    bt = batch_ref[0]
    g = lax.broadcasted_iota(jnp.int32, (1, G), 1)
    oh = (bt == g).astype(jnp.float32)
    dn = (((0,), (0,)), ((), ()))

    @pl.when(i == 0)
    def _():
        sums_ref[...] = jnp.zeros_like(sums_ref)
        sqs_ref[...] = jnp.zeros_like(sqs_ref)
        cnt_ref[...] = jnp.zeros_like(cnt_ref)

    dv = dinv_ref[0]
    for c in range(co2):
        z = dv * agg_ref[c] + b_ref[c]
        sums_ref[c] += lax.dot_general(
            oh, z, dn, preferred_element_type=jnp.float32,
            precision=lax.Precision.HIGHEST)
        sqs_ref[c] += lax.dot_general(
            oh, z * z, dn, preferred_element_type=jnp.float32,
            precision=lax.Precision.HIGHEST)
    cnt_ref[...] += lax.dot_general(
        oh, jnp.ones((1000, 64), jnp.float32), dn,
        preferred_element_type=jnp.float32, precision=lax.Precision.HIGHEST)


def _tc_stats(agg, b3, batch3, dinv3):
    co2 = agg.shape[0]
    return pl.pallas_call(
        functools.partial(_stats_body, co2),
        grid=(10,),
        in_specs=[
            pl.BlockSpec((co2, 1000, 64), lambda i: (0, i, 0)),
            pl.BlockSpec((co2, 1, 64), lambda i: (0, 0, 0)),
            pl.BlockSpec((1, 1000, 1), lambda i: (i, 0, 0)),
            pl.BlockSpec((1, 1000, 1), lambda i: (i, 0, 0)),
        ],
        out_specs=(
            pl.BlockSpec((co2, G, 64), lambda i: (0, 0, 0)),
            pl.BlockSpec((co2, G, 64), lambda i: (0, 0, 0)),
            pl.BlockSpec((G, 64), lambda i: (0, 0)),
        ),
        out_shape=(
            jax.ShapeDtypeStruct((co2, G, 64), jnp.float32),
            jax.ShapeDtypeStruct((co2, G, 64), jnp.float32),
            jax.ShapeDtypeStruct((G, 64), jnp.float32),
        ),
    )(agg, b3, batch3, dinv3)


def _norm_rows(co2, agg_ref, b_ref, batch_ref, dinv_ref, sums_ref,
               sqs_ref, cnt_ref, alpha_ref, gamma_ref, beta_ref, relu):
    n = jnp.maximum(cnt_ref[...], 1.0)
    bt = batch_ref[0]
    g = lax.broadcasted_iota(jnp.int32, (1, G), 1)
    oh = (bt == g).astype(jnp.float32)
    dv = dinv_ref[0]
    ys = []
    for c in range(co2):
        z = dv * agg_ref[c] + b_ref[c]
        m = sums_ref[c] / n
        ex2 = sqs_ref[c] / n
        a = alpha_ref[c]
        var = ex2 - (2.0 * a - a * a) * m * m
        inv = _rsqrt(var + EPS)
        am_row = jnp.dot(oh, a * m, preferred_element_type=jnp.float32,
                         precision=lax.Precision.HIGHEST)
        inv_row = jnp.dot(oh, inv, preferred_element_type=jnp.float32,
                          precision=lax.Precision.HIGHEST)
        y = gamma_ref[c] * (z - am_row) * inv_row + beta_ref[c]
        if relu:
            y = jnp.maximum(y, 0.0)
        ys.append(y)
    return ys


def _norm_mm_body(co2p, co2n, agg_ref, b_ref, batch_ref, dinv_ref, sums_ref,
                  sqs_ref, cnt_ref, alpha_ref, gamma_ref, beta_ref, w_ref,
                  o_ref):
    ys = _norm_rows(co2p, agg_ref, b_ref, batch_ref, dinv_ref, sums_ref,
                    sqs_ref, cnt_ref, alpha_ref, gamma_ref, beta_ref, True)
    dv = dinv_ref[0]
    for co in range(co2n):
        acc = jnp.dot(ys[0], w_ref[0, co], preferred_element_type=jnp.float32)
        for q in range(1, co2p):
            acc += jnp.dot(ys[q], w_ref[q, co],
                           preferred_element_type=jnp.float32)
        o_ref[co] = dv * acc


def _tc_norm_mm(agg, b3, batch3, dinv3, sums, sqs, cnt,
                alpha3, gamma3, beta3, W3r):
    co2p = agg.shape[0]
    co2n = W3r.shape[1]
    return pl.pallas_call(
        functools.partial(_norm_mm_body, co2p, co2n),
        grid=(10,),
        in_specs=[
            pl.BlockSpec((co2p, 1000, 64), lambda i: (0, i, 0)),
            pl.BlockSpec((co2p, 1, 64), lambda i: (0, 0, 0)),
            pl.BlockSpec((1, 1000, 1), lambda i: (i, 0, 0)),
            pl.BlockSpec((1, 1000, 1), lambda i: (i, 0, 0)),
            pl.BlockSpec((co2p, G, 64), lambda i: (0, 0, 0)),
            pl.BlockSpec((co2p, G, 64), lambda i: (0, 0, 0)),
            pl.BlockSpec((G, 64), lambda i: (0, 0)),
            pl.BlockSpec((co2p, 1, 64), lambda i: (0, 0, 0)),
            pl.BlockSpec((co2p, 1, 64), lambda i: (0, 0, 0)),
            pl.BlockSpec((co2p, 1, 64), lambda i: (0, 0, 0)),
            pl.BlockSpec((co2p, co2n, 64, 64), lambda i: (0, 0, 0, 0)),
        ],
        out_specs=pl.BlockSpec((co2n, 1000, 64), lambda i: (0, i, 0)),
        out_shape=jax.ShapeDtypeStruct((co2n, N, 64), jnp.float32),
    )(agg, b3, batch3, dinv3, sums, sqs, cnt, alpha3, gamma3, beta3, W3r)


def _norm_final_body(co2, agg_ref, b_ref, batch_ref, dinv_ref, sums_ref,
                     sqs_ref, cnt_ref, alpha_ref, gamma_ref, beta_ref,
                     out_ref):
    ys = _norm_rows(co2, agg_ref, b_ref, batch_ref, dinv_ref, sums_ref,
                    sqs_ref, cnt_ref, alpha_ref, gamma_ref, beta_ref, False)
    for c in range(co2):
        out_ref[:, pl.ds(c * 64, 64)] = ys[c]


def _tc_norm_final(agg, b3, batch3, dinv3, sums, sqs, cnt,
                   alpha3, gamma3, beta3):
    co2 = agg.shape[0]
    return pl.pallas_call(
        functools.partial(_norm_final_body, co2),
        grid=(10,),
        in_specs=[
            pl.BlockSpec((co2, 1000, 64), lambda i: (0, i, 0)),
            pl.BlockSpec((co2, 1, 64), lambda i: (0, 0, 0)),
            pl.BlockSpec((1, 1000, 1), lambda i: (i, 0, 0)),
            pl.BlockSpec((1, 1000, 1), lambda i: (i, 0, 0)),
            pl.BlockSpec((co2, G, 64), lambda i: (0, 0, 0)),
            pl.BlockSpec((co2, G, 64), lambda i: (0, 0, 0)),
            pl.BlockSpec((G, 64), lambda i: (0, 0)),
            pl.BlockSpec((co2, 1, 64), lambda i: (0, 0, 0)),
            pl.BlockSpec((co2, 1, 64), lambda i: (0, 0, 0)),
            pl.BlockSpec((co2, 1, 64), lambda i: (0, 0, 0)),
        ],
        out_specs=pl.BlockSpec((1000, 64 * co2), lambda i: (i, 0)),
        out_shape=jax.ShapeDtypeStruct((N, 64 * co2), jnp.float32),
    )(agg, b3, batch3, dinv3, sums, sqs, cnt, alpha3, gamma3, beta3)


def kernel(x, edge_index, edge_weight, batch,
           W1, b1, alpha1, gamma1, beta1,
           W2, b2, alpha2, gamma2, beta2,
           W3, b3, alpha3, gamma3, beta3,
           W4, b4, alpha4, gamma4, beta4,
           W5, b5, alpha5, gamma5, beta5):
    src = edge_index[0]
    dst = edge_index[1]

    src2 = src.reshape(16, TROWS, EB)
    dst2 = dst.reshape(16, TROWS, EB)
    w2 = edge_weight.reshape(16, TROWS, EB)
    batch3 = batch.reshape(10, 1000, 1)

    degp = _sc_degree(dst2, w2)
    dinv3 = _tc_dinv(degp)[:, :1].reshape(10, 1000, 1)

    h3 = x.reshape(N, 4, 64).transpose(1, 0, 2)
    layers = [
        (W1, b1, alpha1, gamma1, beta1),
        (W2, b2, alpha2, gamma2, beta2),
        (W3, b3, alpha3, gamma3, beta3),
        (W4, b4, alpha4, gamma4, beta4),
        (W5, b5, alpha5, gamma5, beta5),
    ]

    def wres(W):
        kc = W.shape[0] // 64
        co2 = W.shape[1] // 64
        return W.reshape(kc, 64, co2, 64).transpose(0, 2, 1, 3)

    xw = _tc_matmul(h3, wres(W1), dinv3)
    for li, (W, b, alpha, gamma, beta) in enumerate(layers):
        co2 = W.shape[1] // 64
        agg_fn = _sc_agg8 if co2 == 8 else _sc_agg4
        agg = agg_fn(xw, src2, dst2, w2)
        bc = b.reshape(co2, 1, 64)
        sums, sqs, cnt = _tc_stats(agg, bc, batch3, dinv3)
        params = (bc, batch3, dinv3, sums, sqs, cnt,
                  alpha.reshape(co2, 1, 64), gamma.reshape(co2, 1, 64),
                  beta.reshape(co2, 1, 64))
        if li < 4:
            xw = _tc_norm_mm(agg, *params, wres(layers[li + 1][0]))
        else:
            out = _tc_norm_final(agg, *params)
    return out
